# baseline (speedup 1.0000x reference)
import functools

import jax
import jax.numpy as jnp
from jax import lax
from jax.experimental import pallas as pl
from jax.experimental.pallas import tpu as pltpu
from jax.experimental.pallas import tpu_sc as plsc

_TM = 256
_TN = 512
_TR = 1024
_GCH = 32
_CCH = 16


def _routing_body(n_e, x_ref, gate_wt_ref, ids_ref, ws_ref):
    e_pad = gate_wt_ref.shape[1]
    logits = lax.dot_general(
        x_ref[...], gate_wt_ref[...], (((1,), (0,)), ((), ())),
        preferred_element_type=jnp.float32,
    )
    lane = lax.broadcasted_iota(jnp.int32, logits.shape, 1)
    neg = jnp.float32(-jnp.inf)
    logits = jnp.where(lane < n_e, logits, neg)
    m1 = jnp.max(logits, axis=1)
    i1 = jnp.min(jnp.where(logits == m1[:, None], lane, e_pad), axis=1)
    l2 = jnp.where(lane == i1[:, None], neg, logits)
    m2 = jnp.max(l2, axis=1)
    i2 = jnp.min(jnp.where(l2 == m2[:, None], lane, e_pad), axis=1)
    w1 = 1.0 / (1.0 + jnp.exp(m2 - m1))
    ids_ref[0, :] = i1
    ids_ref[1, :] = i2
    ws_ref[0, :] = w1
    ws_ref[1, :] = 1.0 - w1


def _routing(x2, gate_w):
    t, d_in = x2.shape
    e = gate_w.shape[0]
    gate_wt = jnp.zeros((d_in, 128), jnp.float32).at[:, :e].set(gate_w.T)
    ids, ws = pl.pallas_call(
        functools.partial(_routing_body, e),
        grid=(t // _TR,),
        in_specs=[
            pl.BlockSpec((_TR, d_in), lambda i: (i, 0)),
            pl.BlockSpec((d_in, 128), lambda i: (0, 0)),
        ],
        out_specs=[
            pl.BlockSpec((2, _TR), lambda i: (0, i)),
            pl.BlockSpec((2, _TR), lambda i: (0, i)),
        ],
        out_shape=[
            jax.ShapeDtypeStruct((2, t), jnp.int32),
            jax.ShapeDtypeStruct((2, t), jnp.float32),
        ],
    )(x2, gate_wt)
    return ids, ws


def _dispatch(ids, n_e, n_tiles):
    t = ids.shape[1]
    tk = 2 * t
    ntot = n_tiles * _TM
    e_all = ids.reshape(tk)
    order = jnp.argsort(e_all, stable=True)
    e_sorted = e_all[order]
    counts = jnp.zeros((n_e,), jnp.int32).at[e_all].add(1)
    coff = jnp.concatenate([jnp.zeros((1,), jnp.int32), jnp.cumsum(counts)[:-1]])
    padded = ((counts + _TM - 1) // _TM) * _TM
    pcum = jnp.cumsum(padded)
    poff = jnp.concatenate([jnp.zeros((1,), jnp.int32), pcum[:-1]])
    rank = jnp.arange(tk, dtype=jnp.int32) - coff[e_sorted]
    ppos_sorted = poff[e_sorted] + rank
    ppos = jnp.zeros((tk,), jnp.int32).at[order].set(ppos_sorted)
    gidx = jnp.zeros((ntot,), jnp.int32).at[ppos_sorted].set(order % t)
    tile_expert = jnp.clip(
        jnp.searchsorted(pcum, jnp.arange(n_tiles, dtype=jnp.int32) * _TM,
                         side="right").astype(jnp.int32), 0, n_e - 1)
    return gidx, ppos.reshape(2, t), tile_expert


def _sc_gather(x2, gidx, ntot):
    info = plsc.get_sparse_core_info()
    nc, ns = info.num_cores, info.num_subcores
    nw = nc * ns
    per_w = ntot // nw
    d = x2.shape[1]
    mesh = plsc.VectorSubcoreMesh(core_axis_name="c", subcore_axis_name="s")

    @functools.partial(
        pl.kernel, mesh=mesh,
        out_type=jax.ShapeDtypeStruct((ntot, d), jnp.float32),
        scratch_types=[
            pltpu.VMEM((_GCH,), jnp.int32),
            pltpu.VMEM((_GCH, d), jnp.float32),
            pltpu.SemaphoreType.DMA,
        ],
    )
    def k(xb3_hbm, gidx_hbm, out_hbm, idx_v, rows_v, sem):
        wid = lax.axis_index("s") * nc + lax.axis_index("c")
        base = wid * per_w

        def body(it, carry):
            off = base + it * _GCH
            pltpu.sync_copy(gidx_hbm.at[pl.ds(off, _GCH)], idx_v)
            pltpu.async_copy(xb3_hbm.at[idx_v], rows_v, sem).wait()
            pltpu.sync_copy(rows_v, out_hbm.at[pl.ds(off, _GCH)])
            return carry

        lax.fori_loop(0, per_w // _GCH, body, 0)

    return k(x2, gidx)


def _gmm_body(te_ref, x_ref, wt_ref, b_ref, wc_ref, o_ref):
    y = lax.dot_general(
        x_ref[...].astype(jnp.bfloat16), wt_ref[0], (((1,), (0,)), ((), ())),
        preferred_element_type=jnp.float32,
    )
    o_ref[...] = (y + b_ref[0]) * wc_ref[...]


def _gmm(xs, tile_expert, wt, expert_b3, wcol, n_tiles, d_in, d_out):
    grid = (d_out // _TN, n_tiles)
    return pl.pallas_call(
        _gmm_body,
        grid_spec=pltpu.PrefetchScalarGridSpec(
            num_scalar_prefetch=1,
            grid=grid,
            in_specs=[
                pl.BlockSpec((_TM, d_in), lambda j, i, te: (i, 0)),
                pl.BlockSpec((1, d_in, _TN), lambda j, i, te: (te[i], 0, j)),
                pl.BlockSpec((1, 1, _TN), lambda j, i, te: (te[i], 0, j)),
                pl.BlockSpec((_TM, 1), lambda j, i, te: (i, 0)),
            ],
            out_specs=pl.BlockSpec((_TM, _TN), lambda j, i, te: (i, j)),
        ),
        out_shape=jax.ShapeDtypeStruct((n_tiles * _TM, d_out), jnp.float32),
        compiler_params=pltpu.CompilerParams(
            dimension_semantics=("arbitrary", "arbitrary"),
        ),
    )(tile_expert, xs, wt, expert_b3, wcol)


def _sc_combine(ys, p1, p2, t, d_out):
    info = plsc.get_sparse_core_info()
    nc, ns = info.num_cores, info.num_subcores
    nw = nc * ns
    per_w = t // nw
    mesh = plsc.VectorSubcoreMesh(core_axis_name="c", subcore_axis_name="s")
    nsub = d_out // 16

    @functools.partial(
        pl.kernel, mesh=mesh,
        out_type=jax.ShapeDtypeStruct((t, d_out), jnp.float32),
        scratch_types=[
            pltpu.VMEM((_CCH,), jnp.int32),
            pltpu.VMEM((_CCH,), jnp.int32),
            pltpu.VMEM((_CCH, d_out), jnp.float32),
            pltpu.VMEM((_CCH, d_out), jnp.float32),
            pltpu.SemaphoreType.DMA,
            pltpu.SemaphoreType.DMA,
        ],
    )
    def k(ys_hbm, p1_hbm, p2_hbm, out_hbm, i1v, i2v, buf_a, buf_b, sem_a, sem_b):
        wid = lax.axis_index("s") * nc + lax.axis_index("c")
        base = wid * per_w

        def body(it, carry):
            off = base + it * _CCH
            pltpu.sync_copy(p1_hbm.at[pl.ds(off, _CCH)], i1v)
            pltpu.sync_copy(p2_hbm.at[pl.ds(off, _CCH)], i2v)
            cp_a = pltpu.async_copy(ys_hbm.at[i1v], buf_a, sem_a)
            cp_b = pltpu.async_copy(ys_hbm.at[i2v], buf_b, sem_b)
            cp_a.wait()
            cp_b.wait()

            def row(r, c2):
                for c in range(nsub):
                    sl = pl.ds(c * 16, 16)
                    buf_a[r, sl] = buf_a[r, sl] + buf_b[r, sl]
                return c2

            lax.fori_loop(0, _CCH, row, 0)
            pltpu.sync_copy(buf_a, out_hbm.at[pl.ds(off, _CCH)])
            return carry

        lax.fori_loop(0, per_w // _CCH, body, 0)

    return k(ys, p1, p2)


@jax.jit
def kernel(inputs, gate_w, expert_w, expert_b):
    b, s, d_in = inputs.shape
    n_e, d_out, _ = expert_w.shape
    t = b * s
    n_tiles = (2 * t) // _TM + n_e
    ntot = n_tiles * _TM

    x2 = inputs.reshape(t, d_in)
    wt = jnp.swapaxes(expert_w, 1, 2).astype(jnp.bfloat16)
    expert_b3 = expert_b.reshape(n_e, 1, d_out)

    ids, ws = _routing(x2, gate_w)
    gidx, ppos, tile_expert = _dispatch(ids, n_e, n_tiles)
    wcol = jnp.zeros((ntot, 1), jnp.float32).at[ppos.reshape(-1), 0].set(
        ws.reshape(-1))

    xs = _sc_gather(x2, gidx, ntot)
    ys = _gmm(xs, tile_expert, wt, expert_b3, wcol, n_tiles, d_in, d_out)
    out = _sc_combine(ys, ppos[0], ppos[1], t, d_out)
    return out.reshape(b, s, d_out)
